# baseline (speedup 1.0000x reference)
import jax
import jax.numpy as jnp
from jax import lax
from jax.experimental import pallas as pl
from jax.experimental.pallas import tpu as pltpu

B = 32
N = 512 * 512
SPB = 4
GRID = B // SPB


RB = 8


def _tc_body(x_ref, t_ref, out_ref):
    i = pl.program_id(0)

    for u in range(SPB):
        def step(j, acc):
            s_a, st_a, t_a, b_a = acc
            x = x_ref[u, pl.ds(j * RB, RB), :]
            t = t_ref[u, pl.ds(j * RB, RB), :].astype(jnp.float32)
            ax = jnp.abs(x)
            e = jnp.exp(-ax)
            inv = 1.0 / (1.0 + e)
            sig = jnp.where(x >= 0.0, inv, 1.0 - inv)
            bce = jnp.maximum(x, 0.0) - x * t + jnp.log1p(e)
            return (s_a + sig, st_a + sig * t, t_a + t, b_a + bce)

        z = jnp.zeros((RB, 512), jnp.float32)
        s_a, st_a, t_a, b_a = lax.fori_loop(0, 512 // RB, step, (z, z, z, z))
        out_ref[i * SPB + u, 0] = jnp.sum(s_a)
        out_ref[i * SPB + u, 1] = jnp.sum(st_a)
        out_ref[i * SPB + u, 2] = jnp.sum(t_a)
        out_ref[i * SPB + u, 3] = jnp.sum(b_a)


def _tc_partials(x3, target):
    return pl.pallas_call(
        _tc_body,
        grid=(GRID,),
        in_specs=[
            pl.BlockSpec((SPB, 512, 512), lambda i: (i, 0, 0)),
            pl.BlockSpec((SPB, 512, 512), lambda i: (i, 0, 0)),
        ],
        out_specs=pl.BlockSpec(memory_space=pltpu.SMEM),
        out_shape=jax.ShapeDtypeStruct((B, 4), jnp.float32),
        compiler_params=pltpu.CompilerParams(
            dimension_semantics=("arbitrary",),
        ),
    )(x3, target)


@jax.jit
def kernel(inputs, target):
    x3 = inputs.reshape(B, 512, 512)
    parts = _tc_partials(x3, target)
    s_sum = parts[:, 0]
    st_sum = parts[:, 1]
    t_sum = parts[:, 2]
    b_sum = parts[:, 3]
    dice = 1.0 - (2.0 * st_sum + 1.0) / (s_sum + t_sum + 1.0)
    loss = jnp.mean(dice) + jnp.sum(b_sum) / (B * N)
    return loss.reshape(1)


# speedup vs baseline: 1.2925x; 1.2925x over previous
import jax
import jax.numpy as jnp
from jax import lax
from jax.experimental import pallas as pl
from jax.experimental.pallas import tpu as pltpu

B = 32
N = 512 * 512
SPB = 4
GRID = B // SPB


def _tc_body(x_ref, t_ref, out_ref):
    i = pl.program_id(0)
    x = x_ref[...]
    t = t_ref[...].astype(jnp.float32)

    sig = 0.5 + 0.5 * jnp.tanh(x * 0.5)
    bce = jnp.maximum(x, 0.0) - x * t + jnp.log1p(jnp.exp(-jnp.abs(x)))

    for u in range(SPB):
        out_ref[i * SPB + u, 0] = jnp.sum(sig[u])
        out_ref[i * SPB + u, 1] = jnp.sum(sig[u] * t[u])
        out_ref[i * SPB + u, 2] = jnp.sum(t[u])
        out_ref[i * SPB + u, 3] = jnp.sum(bce[u])


def _tc_partials(x3, target):
    return pl.pallas_call(
        _tc_body,
        grid=(GRID,),
        in_specs=[
            pl.BlockSpec((SPB, 512, 512), lambda i: (i, 0, 0)),
            pl.BlockSpec((SPB, 512, 512), lambda i: (i, 0, 0)),
        ],
        out_specs=pl.BlockSpec(memory_space=pltpu.SMEM),
        out_shape=jax.ShapeDtypeStruct((B, 4), jnp.float32),
        compiler_params=pltpu.CompilerParams(
            dimension_semantics=("arbitrary",),
        ),
    )(x3, target)


@jax.jit
def kernel(inputs, target):
    x3 = inputs.reshape(B, 512, 512)
    parts = _tc_partials(x3, target)
    s_sum = parts[:, 0]
    st_sum = parts[:, 1]
    t_sum = parts[:, 2]
    b_sum = parts[:, 3]
    dice = 1.0 - (2.0 * st_sum + 1.0) / (s_sum + t_sum + 1.0)
    loss = jnp.mean(dice) + jnp.sum(b_sum) / (B * N)
    return loss.reshape(1)
